# baseline (speedup 1.0000x reference)
import jax
import jax.numpy as jnp
from jax import lax
from jax.experimental import pallas as pl
from jax.experimental.pallas import tpu as pltpu

CLS_W = 10.0
REG_W = 8.0
GAMMA = 2.0
ALPHA = 0.25

B, T, M, FW, D = 256, 128, 20, 8, 3
ROW = FW * D
GB = 16
GRID = B // GB
INF = float("inf")


def _body(a3_ref, px_ref, py_ref, pz_ref, cls0_ref, cls1_ref, reg0_ref, reg1_ref,
          c0_ref, c1_ref, r0_ref, r1_ref):
    pid = pl.program_id(0)
    xx = px_ref[...]
    xy = py_ref[...]
    xz = pz_ref[...]

    txs = [xx[:, f, :] for f in range(FW)]
    tys = [xy[:, f, :] for f in range(FW)]
    tzs = [xz[:, f, :] for f in range(FW)]
    targ = jnp.concatenate(
        [v[:, None, :] for v in txs + tys + tzs], axis=1)

    txy1 = jnp.concatenate(
        [targ[:, 0:16, :], jnp.ones((GB, 1, 128), jnp.float32)], axis=1)
    a3 = jnp.broadcast_to(a3_ref[...][None], (GB, M, 17))
    dist2 = lax.dot_general(
        a3, txy1, (((2,), (1,)), ((0,), (0,))),
        preferred_element_type=jnp.float32)

    ii = lax.broadcasted_iota(jnp.int32, (GB, 8, 128), 1)
    v = dist2[:, 0:8, :]
    mi = ii
    d1 = dist2[:, 8:16, :]
    u = d1 < v
    v = jnp.where(u, d1, v)
    mi = jnp.where(u, ii + 8, mi)
    d2 = jnp.concatenate(
        [dist2[:, 16:20, :], jnp.full((GB, 4, 128), INF)], axis=1)
    u = d2 < v
    v = jnp.where(u, d2, v)
    mi = jnp.where(u, ii + 16, mi)
    for sh in (4, 2, 1):
        vr = jnp.concatenate([v[:, sh:, :], v[:, :sh, :]], axis=1)
        mir = jnp.concatenate([mi[:, sh:, :], mi[:, :sh, :]], axis=1)
        u = (vr < v) | ((vr == v) & (mir < mi))
        v = jnp.where(u, vr, v)
        mi = jnp.where(u, mir, mi)

    mode = jnp.concatenate([mi[b, 0:1, :] for b in range(GB)], axis=0)

    masks = [mode == m for m in range(M)]

    def focal_sum(cls_blk):
        sacc = jnp.zeros((GB, 128), jnp.float32)
        hot = cls_blk[0]
        for m in range(M):
            pred = cls_blk[m]
            p = jax.nn.sigmoid(pred)
            sp = (jnp.maximum(pred, 0.0)
                  + jnp.log1p(jnp.exp(-jnp.abs(pred))))
            sacc = sacc + ((1.0 - ALPHA) * p * p) * sp
            if m > 0:
                hot = jnp.where(masks[m], pred, hot)
        ph = jax.nn.sigmoid(hot)
        sph = jnp.maximum(hot, 0.0) + jnp.log1p(jnp.exp(-jnp.abs(hot)))
        corr = (ALPHA * (1.0 - ph) * (1.0 - ph)) * (sph - hot) \
            - ((1.0 - ALPHA) * ph * ph) * sph
        return jnp.sum(sacc) + jnp.sum(corr)

    s_c0 = focal_sum(cls0_ref[...])
    s_c1 = focal_sum(cls1_ref[...])

    modeb = jnp.broadcast_to(mode[:, None, :], (GB, ROW, 128))
    masksb = [modeb == m for m in range(1, M)]

    def reg_sum(reg_blk):
        sel = reg_blk[:, 0, :, :]
        for m in range(1, M):
            sel = jnp.where(masksb[m - 1], reg_blk[:, m, :, :], sel)
        return jnp.sum(jnp.abs(sel - targ))

    s_r0 = reg_sum(reg0_ref[...])
    s_r1 = reg_sum(reg1_ref[...])

    @pl.when(pid == 0)
    def _():
        c0_ref[...] = jnp.zeros_like(c0_ref)
        c1_ref[...] = jnp.zeros_like(c1_ref)
        r0_ref[...] = jnp.zeros_like(r0_ref)
        r1_ref[...] = jnp.zeros_like(r1_ref)

    c0_ref[...] += s_c0[None, None]
    c1_ref[...] += s_c1[None, None]
    r0_ref[...] += s_r0[None, None]
    r1_ref[...] += s_r1[None, None]


def kernel(diff_traj_reg_0, diff_traj_cls_0, diff_traj_reg_1,
           diff_traj_cls_1, future_ego_n_to_ego_curr, anchor):
    posesv = future_ego_n_to_ego_curr.transpose(0, 2, 3, 4, 1).reshape(
        B, FW, 16, T)
    px = posesv[:, :, 3, :]
    py = posesv[:, :, 7, :]
    pz = posesv[:, :, 11, :]
    cls0v = diff_traj_cls_0.transpose(2, 0, 1)
    cls1v = diff_traj_cls_1.transpose(2, 0, 1)
    reg0v = diff_traj_reg_0.transpose(0, 2, 4, 3, 1).reshape(B, M, ROW, T)
    reg1v = diff_traj_reg_1.transpose(0, 2, 4, 3, 1).reshape(B, M, ROW, T)

    a2 = anchor.reshape(M, 2 * FW)
    a2p = jnp.concatenate([a2[:, 0::2], a2[:, 1::2]], axis=1)
    anorm = jnp.sum(a2 * a2, axis=1, keepdims=True)
    a3 = jnp.concatenate([-2.0 * a2p, anorm], axis=1)

    acc1x1 = [
        pl.BlockSpec((1, 1), lambda i: (0, 0)),
        pl.BlockSpec((1, 1), lambda i: (0, 0)),
        pl.BlockSpec((1, 1), lambda i: (0, 0)),
        pl.BlockSpec((1, 1), lambda i: (0, 0)),
    ]
    c0, c1, r0, r1 = pl.pallas_call(
        _body,
        grid=(GRID,),
        in_specs=[
            pl.BlockSpec((M, 17), lambda i: (0, 0)),
            pl.BlockSpec((GB, FW, T), lambda i: (i, 0, 0)),
            pl.BlockSpec((GB, FW, T), lambda i: (i, 0, 0)),
            pl.BlockSpec((GB, FW, T), lambda i: (i, 0, 0)),
            pl.BlockSpec((M, GB, T), lambda i: (0, i, 0)),
            pl.BlockSpec((M, GB, T), lambda i: (0, i, 0)),
            pl.BlockSpec((GB, M, ROW, T), lambda i: (i, 0, 0, 0)),
            pl.BlockSpec((GB, M, ROW, T), lambda i: (i, 0, 0, 0)),
        ],
        out_specs=acc1x1,
        out_shape=[jax.ShapeDtypeStruct((1, 1), jnp.float32)] * 4,
        compiler_params=pltpu.CompilerParams(
            dimension_semantics=("arbitrary",)),
    )(a3, px, py, pz, cls0v, cls1v, reg0v, reg1v)

    cls_loss = (c0[0, 0] + c1[0, 0]) / (B * T * M)
    reg_loss = (r0[0, 0] + r1[0, 0]) / (B * T * ROW)
    return CLS_W * cls_loss + REG_W * reg_loss


# speedup vs baseline: 1.0259x; 1.0259x over previous
import jax
import jax.numpy as jnp
from jax import lax
from jax.experimental import pallas as pl
from jax.experimental.pallas import tpu as pltpu

CLS_W = 10.0
REG_W = 8.0
GAMMA = 2.0
ALPHA = 0.25

B, T, M, FW, D = 256, 128, 20, 8, 3
ROW = FW * D
GB = 16
GRID = B // GB
INF = float("inf")


def _body(a3_ref, poses_hbm, cls0_ref, cls1_ref, reg0_ref, reg1_ref,
          c0_ref, c1_ref, r0_ref, r1_ref, xb_ref, yb_ref, zb_ref, sem):
    pid = pl.program_id(0)
    bs = pl.ds(pid * GB, GB)
    cpx = pltpu.make_async_copy(
        poses_hbm.at[bs, :, pl.ds(3, 1), :, :], xb_ref, sem)
    cpy = pltpu.make_async_copy(
        poses_hbm.at[bs, :, pl.ds(7, 1), :, :], yb_ref, sem)
    cpz = pltpu.make_async_copy(
        poses_hbm.at[bs, :, pl.ds(11, 1), :, :], zb_ref, sem)
    cpx.start()
    cpy.start()
    cpz.start()
    cpx.wait()
    cpy.wait()
    cpz.wait()
    xx = xb_ref[...]
    xy = yb_ref[...]
    xz = zb_ref[...]

    txs = [xx[:, f, 0, 0, :] for f in range(FW)]
    tys = [xy[:, f, 0, 0, :] for f in range(FW)]
    tzs = [xz[:, f, 0, 0, :] for f in range(FW)]
    targ = jnp.concatenate(
        [v[:, None, :] for v in txs + tys + tzs], axis=1)

    txy1 = jnp.concatenate(
        [targ[:, 0:16, :], jnp.ones((GB, 1, 128), jnp.float32)], axis=1)
    a3 = jnp.broadcast_to(a3_ref[...][None], (GB, M, 17))
    dist2 = lax.dot_general(
        a3, txy1, (((2,), (1,)), ((0,), (0,))),
        preferred_element_type=jnp.float32)

    ii = lax.broadcasted_iota(jnp.int32, (GB, 8, 128), 1)
    v = dist2[:, 0:8, :]
    mi = ii
    d1 = dist2[:, 8:16, :]
    u = d1 < v
    v = jnp.where(u, d1, v)
    mi = jnp.where(u, ii + 8, mi)
    d2 = jnp.concatenate(
        [dist2[:, 16:20, :], jnp.full((GB, 4, 128), INF)], axis=1)
    u = d2 < v
    v = jnp.where(u, d2, v)
    mi = jnp.where(u, ii + 16, mi)
    for sh in (4, 2, 1):
        vr = jnp.concatenate([v[:, sh:, :], v[:, :sh, :]], axis=1)
        mir = jnp.concatenate([mi[:, sh:, :], mi[:, :sh, :]], axis=1)
        u = (vr < v) | ((vr == v) & (mir < mi))
        v = jnp.where(u, vr, v)
        mi = jnp.where(u, mir, mi)

    mode = jnp.concatenate([mi[b, 0:1, :] for b in range(GB)], axis=0)

    masks = [mode == m for m in range(M)]

    def focal_sum(cls_blk):
        sacc = jnp.zeros((GB, 128), jnp.float32)
        hot = cls_blk[0]
        for m in range(M):
            pred = cls_blk[m]
            p = jax.nn.sigmoid(pred)
            sp = (jnp.maximum(pred, 0.0)
                  + jnp.log1p(jnp.exp(-jnp.abs(pred))))
            sacc = sacc + ((1.0 - ALPHA) * p * p) * sp
            if m > 0:
                hot = jnp.where(masks[m], pred, hot)
        ph = jax.nn.sigmoid(hot)
        sph = jnp.maximum(hot, 0.0) + jnp.log1p(jnp.exp(-jnp.abs(hot)))
        corr = (ALPHA * (1.0 - ph) * (1.0 - ph)) * (sph - hot) \
            - ((1.0 - ALPHA) * ph * ph) * sph
        return jnp.sum(sacc) + jnp.sum(corr)

    s_c0 = focal_sum(cls0_ref[...])
    s_c1 = focal_sum(cls1_ref[...])

    modeb = jnp.broadcast_to(mode[:, None, :], (GB, ROW, 128))
    masksb = [modeb == m for m in range(1, M)]

    def reg_sum(reg_blk):
        sel = reg_blk[:, 0, :, :]
        for m in range(1, M):
            sel = jnp.where(masksb[m - 1], reg_blk[:, m, :, :], sel)
        return jnp.sum(jnp.abs(sel - targ))

    s_r0 = reg_sum(reg0_ref[...])
    s_r1 = reg_sum(reg1_ref[...])

    @pl.when(pid == 0)
    def _():
        c0_ref[...] = jnp.zeros_like(c0_ref)
        c1_ref[...] = jnp.zeros_like(c1_ref)
        r0_ref[...] = jnp.zeros_like(r0_ref)
        r1_ref[...] = jnp.zeros_like(r1_ref)

    c0_ref[...] += s_c0[None, None]
    c1_ref[...] += s_c1[None, None]
    r0_ref[...] += s_r0[None, None]
    r1_ref[...] += s_r1[None, None]


def kernel(diff_traj_reg_0, diff_traj_cls_0, diff_traj_reg_1,
           diff_traj_cls_1, future_ego_n_to_ego_curr, anchor):
    posesv = future_ego_n_to_ego_curr.transpose(0, 2, 3, 4, 1).reshape(
        B, FW, 16, 1, T)
    cls0v = diff_traj_cls_0.transpose(2, 0, 1)
    cls1v = diff_traj_cls_1.transpose(2, 0, 1)
    reg0v = diff_traj_reg_0.transpose(0, 2, 4, 3, 1).reshape(B, M, ROW, T)
    reg1v = diff_traj_reg_1.transpose(0, 2, 4, 3, 1).reshape(B, M, ROW, T)

    a2 = anchor.reshape(M, 2 * FW)
    a2p = jnp.concatenate([a2[:, 0::2], a2[:, 1::2]], axis=1)
    anorm = jnp.sum(a2 * a2, axis=1, keepdims=True)
    a3 = jnp.concatenate([-2.0 * a2p, anorm], axis=1)

    acc1x1 = [
        pl.BlockSpec((1, 1), lambda i: (0, 0)),
        pl.BlockSpec((1, 1), lambda i: (0, 0)),
        pl.BlockSpec((1, 1), lambda i: (0, 0)),
        pl.BlockSpec((1, 1), lambda i: (0, 0)),
    ]
    c0, c1, r0, r1 = pl.pallas_call(
        _body,
        grid=(GRID,),
        in_specs=[
            pl.BlockSpec((M, 17), lambda i: (0, 0)),
            pl.BlockSpec(memory_space=pltpu.MemorySpace.HBM),
            pl.BlockSpec((M, GB, T), lambda i: (0, i, 0)),
            pl.BlockSpec((M, GB, T), lambda i: (0, i, 0)),
            pl.BlockSpec((GB, M, ROW, T), lambda i: (i, 0, 0, 0)),
            pl.BlockSpec((GB, M, ROW, T), lambda i: (i, 0, 0, 0)),
        ],
        out_specs=acc1x1,
        out_shape=[jax.ShapeDtypeStruct((1, 1), jnp.float32)] * 4,
        scratch_shapes=[
            pltpu.VMEM((GB, FW, 1, 1, T), jnp.float32),
            pltpu.VMEM((GB, FW, 1, 1, T), jnp.float32),
            pltpu.VMEM((GB, FW, 1, 1, T), jnp.float32),
            pltpu.SemaphoreType.DMA,
        ],
        compiler_params=pltpu.CompilerParams(
            dimension_semantics=("arbitrary",)),
    )(a3, posesv, cls0v, cls1v, reg0v, reg1v)

    cls_loss = (c0[0, 0] + c1[0, 0]) / (B * T * M)
    reg_loss = (r0[0, 0] + r1[0, 0]) / (B * T * ROW)
    return CLS_W * cls_loss + REG_W * reg_loss


# speedup vs baseline: 1.6905x; 1.6478x over previous
import jax
import jax.numpy as jnp
from jax import lax
from jax.experimental import pallas as pl
from jax.experimental.pallas import tpu as pltpu

CLS_W = 10.0
REG_W = 8.0
GAMMA = 2.0
ALPHA = 0.25

B, T, M, FW, D = 256, 128, 20, 8, 3
ROW = FW * D
GB = 16
GRID = B // GB
INF = float("inf")


def _body(a3_ref, px_ref, py_ref, pz_ref, cls0_ref, cls1_ref, reg0_ref, reg1_ref,
          c0_ref, c1_ref, r0_ref, r1_ref):
    pid = pl.program_id(0)
    xx = px_ref[...]
    xy = py_ref[...]
    xz = pz_ref[...]

    txs = [xx[:, f, 0, 0, :] for f in range(FW)]
    tys = [xy[:, f, 0, 0, :] for f in range(FW)]
    tzs = [xz[:, f, 0, 0, :] for f in range(FW)]
    targ = jnp.concatenate(
        [v[:, None, :] for v in txs + tys + tzs], axis=1)

    txy1 = jnp.concatenate(
        [targ[:, 0:16, :], jnp.ones((GB, 1, 128), jnp.float32)], axis=1)
    a3 = jnp.broadcast_to(a3_ref[...][None], (GB, M, 17))
    dist2 = lax.dot_general(
        a3, txy1, (((2,), (1,)), ((0,), (0,))),
        preferred_element_type=jnp.float32)

    ii = lax.broadcasted_iota(jnp.int32, (GB, 8, 128), 1)
    v = dist2[:, 0:8, :]
    mi = ii
    d1 = dist2[:, 8:16, :]
    u = d1 < v
    v = jnp.where(u, d1, v)
    mi = jnp.where(u, ii + 8, mi)
    d2 = jnp.concatenate(
        [dist2[:, 16:20, :], jnp.full((GB, 4, 128), INF)], axis=1)
    u = d2 < v
    v = jnp.where(u, d2, v)
    mi = jnp.where(u, ii + 16, mi)
    for sh in (4, 2, 1):
        vr = jnp.concatenate([v[:, sh:, :], v[:, :sh, :]], axis=1)
        mir = jnp.concatenate([mi[:, sh:, :], mi[:, :sh, :]], axis=1)
        u = (vr < v) | ((vr == v) & (mir < mi))
        v = jnp.where(u, vr, v)
        mi = jnp.where(u, mir, mi)

    mode = jnp.concatenate([mi[b, 0:1, :] for b in range(GB)], axis=0)

    masks = [mode == m for m in range(M)]

    def focal_sum(cls_blk):
        sacc = jnp.zeros((GB, 128), jnp.float32)
        hot = cls_blk[0]
        for m in range(M):
            pred = cls_blk[m]
            p = jax.nn.sigmoid(pred)
            sp = (jnp.maximum(pred, 0.0)
                  + jnp.log1p(jnp.exp(-jnp.abs(pred))))
            sacc = sacc + ((1.0 - ALPHA) * p * p) * sp
            if m > 0:
                hot = jnp.where(masks[m], pred, hot)
        ph = jax.nn.sigmoid(hot)
        sph = jnp.maximum(hot, 0.0) + jnp.log1p(jnp.exp(-jnp.abs(hot)))
        corr = (ALPHA * (1.0 - ph) * (1.0 - ph)) * (sph - hot) \
            - ((1.0 - ALPHA) * ph * ph) * sph
        return jnp.sum(sacc) + jnp.sum(corr)

    s_c0 = focal_sum(cls0_ref[...])
    s_c1 = focal_sum(cls1_ref[...])

    modeb = jnp.broadcast_to(mode[:, None, :], (GB, ROW, 128))
    masksb = [modeb == m for m in range(1, M)]

    def reg_sum(reg_blk):
        sel = reg_blk[:, 0, :, :]
        for m in range(1, M):
            sel = jnp.where(masksb[m - 1], reg_blk[:, m, :, :], sel)
        return jnp.sum(jnp.abs(sel - targ))

    s_r0 = reg_sum(reg0_ref[...])
    s_r1 = reg_sum(reg1_ref[...])

    @pl.when(pid == 0)
    def _():
        c0_ref[...] = jnp.zeros_like(c0_ref)
        c1_ref[...] = jnp.zeros_like(c1_ref)
        r0_ref[...] = jnp.zeros_like(r0_ref)
        r1_ref[...] = jnp.zeros_like(r1_ref)

    c0_ref[...] += s_c0[None, None]
    c1_ref[...] += s_c1[None, None]
    r0_ref[...] += s_r0[None, None]
    r1_ref[...] += s_r1[None, None]


def kernel(diff_traj_reg_0, diff_traj_cls_0, diff_traj_reg_1,
           diff_traj_cls_1, future_ego_n_to_ego_curr, anchor):
    posesv = future_ego_n_to_ego_curr.transpose(0, 2, 3, 4, 1).reshape(
        B, FW, 16, 1, T)
    cls0v = diff_traj_cls_0.transpose(2, 0, 1)
    cls1v = diff_traj_cls_1.transpose(2, 0, 1)
    reg0v = diff_traj_reg_0.transpose(0, 2, 4, 3, 1).reshape(B, M, ROW, T)
    reg1v = diff_traj_reg_1.transpose(0, 2, 4, 3, 1).reshape(B, M, ROW, T)

    a2 = anchor.reshape(M, 2 * FW)
    a2p = jnp.concatenate([a2[:, 0::2], a2[:, 1::2]], axis=1)
    anorm = jnp.sum(a2 * a2, axis=1, keepdims=True)
    a3 = jnp.concatenate([-2.0 * a2p, anorm], axis=1)

    acc1x1 = [
        pl.BlockSpec((1, 1), lambda i: (0, 0)),
        pl.BlockSpec((1, 1), lambda i: (0, 0)),
        pl.BlockSpec((1, 1), lambda i: (0, 0)),
        pl.BlockSpec((1, 1), lambda i: (0, 0)),
    ]
    c0, c1, r0, r1 = pl.pallas_call(
        _body,
        grid=(GRID,),
        in_specs=[
            pl.BlockSpec((M, 17), lambda i: (0, 0)),
            pl.BlockSpec((GB, FW, 1, 1, T), lambda i: (i, 0, 3, 0, 0)),
            pl.BlockSpec((GB, FW, 1, 1, T), lambda i: (i, 0, 7, 0, 0)),
            pl.BlockSpec((GB, FW, 1, 1, T), lambda i: (i, 0, 11, 0, 0)),
            pl.BlockSpec((M, GB, T), lambda i: (0, i, 0)),
            pl.BlockSpec((M, GB, T), lambda i: (0, i, 0)),
            pl.BlockSpec((GB, M, ROW, T), lambda i: (i, 0, 0, 0)),
            pl.BlockSpec((GB, M, ROW, T), lambda i: (i, 0, 0, 0)),
        ],
        out_specs=acc1x1,
        out_shape=[jax.ShapeDtypeStruct((1, 1), jnp.float32)] * 4,
        compiler_params=pltpu.CompilerParams(
            dimension_semantics=("arbitrary",)),
    )(a3, posesv, posesv, posesv, cls0v, cls1v, reg0v, reg1v)

    cls_loss = (c0[0, 0] + c1[0, 0]) / (B * T * M)
    reg_loss = (r0[0, 0] + r1[0, 0]) / (B * T * ROW)
    return CLS_W * cls_loss + REG_W * reg_loss
